# baseline (speedup 1.0000x reference)
import functools

import jax
import jax.numpy as jnp
from jax import lax
from jax.experimental import pallas as pl
from jax.experimental.pallas import tpu as pltpu
from jax.experimental.pallas import tpu_sc as plsc

R = 64
N = 1_000_000
L = 16
NC = 2
NS = 16
NW = NC * NS
CCOLS = 1024
UPR = 976
NRG = R // 8
UNITS = NRG * UPR
UPW = UNITS // NW
DEPTH = 4
UNROLL = 8
INNER = CCOLS // L // UNROLL
TCOLS = 576
TVALID = N - UPR * CCOLS
BIG_I32 = 2147483647

_mesh = plsc.VectorSubcoreMesh(core_axis_name="c", subcore_axis_name="s")


@functools.partial(
    pl.kernel,
    out_type=(jax.ShapeDtypeStruct((NW, R, L), jnp.float32),
              jax.ShapeDtypeStruct((NW, R, L), jnp.int32)),
    mesh=_mesh,
    scratch_types=(
        [pltpu.VMEM((8, CCOLS), jnp.float32) for _ in range(2 * DEPTH)]
        + [pltpu.VMEM((8, TCOLS), jnp.float32) for _ in range(2)]
        + [pltpu.VMEM((R, L), jnp.float32), pltpu.VMEM((R, L), jnp.int32)]
        + [pltpu.SemaphoreType.DMA for _ in range(2 * DEPTH)]
    ),
)
def _sc_argmax(scores_hbm, gumbel_hbm, outm_hbm, outi_hbm, *scratch):
    sbufs = scratch[:DEPTH]
    gbufs = scratch[DEPTH:2 * DEPTH]
    ts, tg = scratch[2 * DEPTH], scratch[2 * DEPTH + 1]
    acc_m = scratch[2 * DEPTH + 2]
    acc_i = scratch[2 * DEPTH + 3]
    sems_s = scratch[2 * DEPTH + 4:2 * DEPTH + 4 + DEPTH]
    sems_g = scratch[2 * DEPTH + 4 + DEPTH:]

    wid = lax.axis_index("s") * NC + lax.axis_index("c")
    lane = lax.iota(jnp.int32, L)

    neg_inf = jnp.full((L,), -jnp.inf, jnp.float32)
    zeros_i = jnp.zeros((L,), jnp.int32)
    for row in range(R):
        acc_m.at[row][...] = neg_inf
        acc_i.at[row][...] = zeros_i

    def unit_src(arr, u):
        rg = u // UPR
        cp = u - rg * UPR
        ro = pl.multiple_of(rg * 8, 8)
        co = pl.multiple_of(cp * CCOLS, CCOLS)
        return arr.at[pl.ds(ro, 8), pl.ds(co, CCOLS)]

    for b in range(DEPTH):
        pltpu.async_copy(unit_src(scores_hbm, wid + NW * b), sbufs[b], sems_s[b])
        pltpu.async_copy(unit_src(gumbel_hbm, wid + NW * b), gbufs[b], sems_g[b])

    def process_slab(sb, gb, rg, col0, nvec, masked):
        row0 = rg * 8
        for rr in range(8):
            row = row0 + rr
            m = acc_m[row]
            mi = acc_i[row]
            idxv0 = col0 + lane

            if masked:
                def step(i, car, sb=sb, gb=gb, rr=rr):
                    m, mi, idxv = car
                    off = pl.multiple_of(i * L, L)
                    v = sb[rr, pl.ds(off, L)] + gb[rr, pl.ds(off, L)]
                    upd = (v > m) & (idxv < N)
                    m = jnp.where(upd, v, m)
                    mi = jnp.where(upd, idxv, mi)
                    return m, mi, idxv + L
                m, mi, _ = lax.fori_loop(0, nvec, step, (m, mi, idxv0))
            else:
                def step(i, car, sb=sb, gb=gb, rr=rr):
                    m, mi, idxv = car
                    base = pl.multiple_of(i * (UNROLL * L), UNROLL * L)
                    for uu in range(UNROLL):
                        off = base + uu * L
                        v = sb[rr, pl.ds(off, L)] + gb[rr, pl.ds(off, L)]
                        upd = v > m
                        m = jnp.where(upd, v, m)
                        mi = jnp.where(upd, idxv + uu * L, mi)
                    return m, mi, idxv + UNROLL * L
                m, mi, _ = lax.fori_loop(0, nvec // UNROLL, step, (m, mi, idxv0))

            acc_m.at[row][...] = m
            acc_i.at[row][...] = mi

    def ring_step(t2, carry):
        for b in range(DEPTH):
            sb, gb, ss, gs = sbufs[b], gbufs[b], sems_s[b], sems_g[b]
            t = t2 * DEPTH + b
            pltpu.make_async_copy(unit_src(scores_hbm, 0), sb, ss).wait()
            pltpu.make_async_copy(unit_src(gumbel_hbm, 0), gb, gs).wait()

            u = wid + NW * t
            rg = u // UPR
            cp = u - rg * UPR
            process_slab(sb, gb, rg, cp * CCOLS, CCOLS // L, masked=False)

            @pl.when(t + DEPTH < UPW)
            def _(sb=sb, gb=gb, ss=ss, gs=gs, t=t):
                un = wid + NW * (t + DEPTH)
                pltpu.async_copy(unit_src(scores_hbm, un), sb, ss)
                pltpu.async_copy(unit_src(gumbel_hbm, un), gb, gs)
        return carry

    lax.fori_loop(0, UPW // DEPTH, ring_step, 0)

    @pl.when(wid < NRG)
    def _():
        ro = pl.multiple_of(wid * 8, 8)
        co = UPR * CCOLS
        pltpu.sync_copy(scores_hbm.at[pl.ds(ro, 8), pl.ds(co, TCOLS)], ts)
        pltpu.sync_copy(gumbel_hbm.at[pl.ds(ro, 8), pl.ds(co, TCOLS)], tg)
        process_slab(ts, tg, wid, co, TCOLS // L, masked=True)

    pltpu.sync_copy(acc_m, outm_hbm.at[wid])
    pltpu.sync_copy(acc_i, outi_hbm.at[wid])


def kernel(scores, gumbel):
    outm, outi = _sc_argmax(scores, gumbel)
    m = outm.transpose(1, 0, 2).reshape(R, NW * L)
    mi = outi.transpose(1, 0, 2).reshape(R, NW * L)
    gmax = jnp.max(m, axis=1, keepdims=True)
    gidx = jnp.min(jnp.where(m == gmax, mi, BIG_I32), axis=1)
    return gidx[:, None].astype(jnp.int32)


# speedup vs baseline: 1.0858x; 1.0858x over previous
import functools

import jax
import jax.numpy as jnp
from jax import lax
from jax.experimental import pallas as pl
from jax.experimental.pallas import tpu as pltpu
from jax.experimental.pallas import tpu_sc as plsc

R = 64
N = 1_000_000
L = 16
NC = 2
NS = 16
NW = NC * NS
CCOLS = 2048
UPR = 488
NRG = R // 8
UNITS = NRG * UPR
UPW = UNITS // NW
DEPTH = 2
UNROLL = 8
INNER = CCOLS // L // UNROLL
TCOLS = 576
TVALID = N - UPR * CCOLS
BIG_I32 = 2147483647

_mesh = plsc.VectorSubcoreMesh(core_axis_name="c", subcore_axis_name="s")


@functools.partial(
    pl.kernel,
    out_type=(jax.ShapeDtypeStruct((NW, R, L), jnp.float32),
              jax.ShapeDtypeStruct((NW, R, L), jnp.int32)),
    mesh=_mesh,
    scratch_types=(
        [pltpu.VMEM((8, CCOLS), jnp.float32) for _ in range(2 * DEPTH)]
        + [pltpu.VMEM((8, TCOLS), jnp.float32) for _ in range(2)]
        + [pltpu.VMEM((R, L), jnp.float32), pltpu.VMEM((R, L), jnp.int32)]
        + [pltpu.SemaphoreType.DMA for _ in range(2 * DEPTH)]
    ),
)
def _sc_argmax(scores_hbm, gumbel_hbm, outm_hbm, outi_hbm, *scratch):
    sbufs = scratch[:DEPTH]
    gbufs = scratch[DEPTH:2 * DEPTH]
    ts, tg = scratch[2 * DEPTH], scratch[2 * DEPTH + 1]
    acc_m = scratch[2 * DEPTH + 2]
    acc_i = scratch[2 * DEPTH + 3]
    sems_s = scratch[2 * DEPTH + 4:2 * DEPTH + 4 + DEPTH]
    sems_g = scratch[2 * DEPTH + 4 + DEPTH:]

    wid = lax.axis_index("s") * NC + lax.axis_index("c")
    lane = lax.iota(jnp.int32, L)

    neg_inf = jnp.full((L,), -jnp.inf, jnp.float32)
    zeros_i = jnp.zeros((L,), jnp.int32)
    for row in range(R):
        acc_m.at[row][...] = neg_inf
        acc_i.at[row][...] = zeros_i

    def unit_src(arr, u):
        rg = u // UPR
        cp = u - rg * UPR
        ro = pl.multiple_of(rg * 8, 8)
        co = pl.multiple_of(cp * CCOLS, CCOLS)
        return arr.at[pl.ds(ro, 8), pl.ds(co, CCOLS)]

    for b in range(DEPTH):
        pltpu.async_copy(unit_src(scores_hbm, wid * UPW + b), sbufs[b], sems_s[b])
        pltpu.async_copy(unit_src(gumbel_hbm, wid * UPW + b), gbufs[b], sems_g[b])

    def process_slab(sb, gb, rg, col0, nvec, masked):
        row0 = rg * 8
        for rr in range(8):
            row = row0 + rr
            m = acc_m[row]
            mi = acc_i[row]
            idxv0 = col0 + lane

            if masked:
                def step(i, car, sb=sb, gb=gb, rr=rr):
                    m, mi, idxv = car
                    off = pl.multiple_of(i * L, L)
                    v = sb[rr, pl.ds(off, L)] + gb[rr, pl.ds(off, L)]
                    upd = (v > m) & (idxv < N)
                    m = jnp.where(upd, v, m)
                    mi = jnp.where(upd, idxv, mi)
                    return m, mi, idxv + L
                m, mi, _ = lax.fori_loop(0, nvec, step, (m, mi, idxv0))
            else:
                def step(i, car, sb=sb, gb=gb, rr=rr):
                    m, mi, idxv = car
                    base = pl.multiple_of(i * (UNROLL * L), UNROLL * L)
                    for uu in range(UNROLL):
                        off = base + uu * L
                        v = sb[rr, pl.ds(off, L)] + gb[rr, pl.ds(off, L)]
                        upd = v > m
                        m = jnp.where(upd, v, m)
                        mi = jnp.where(upd, idxv + uu * L, mi)
                    return m, mi, idxv + UNROLL * L
                m, mi, _ = lax.fori_loop(0, nvec // UNROLL, step, (m, mi, idxv0))

            acc_m.at[row][...] = m
            acc_i.at[row][...] = mi

    def ring_step(t2, carry):
        for b in range(DEPTH):
            sb, gb, ss, gs = sbufs[b], gbufs[b], sems_s[b], sems_g[b]
            t = t2 * DEPTH + b
            pltpu.make_async_copy(unit_src(scores_hbm, 0), sb, ss).wait()
            pltpu.make_async_copy(unit_src(gumbel_hbm, 0), gb, gs).wait()

            u = wid * UPW + t
            rg = u // UPR
            cp = u - rg * UPR
            process_slab(sb, gb, rg, cp * CCOLS, CCOLS // L, masked=False)

            @pl.when(t + DEPTH < UPW)
            def _(sb=sb, gb=gb, ss=ss, gs=gs, t=t):
                un = wid * UPW + t + DEPTH
                pltpu.async_copy(unit_src(scores_hbm, un), sb, ss)
                pltpu.async_copy(unit_src(gumbel_hbm, un), gb, gs)
        return carry

    lax.fori_loop(0, UPW // DEPTH, ring_step, 0)

    @pl.when(wid < NRG)
    def _():
        ro = pl.multiple_of(wid * 8, 8)
        co = UPR * CCOLS
        pltpu.sync_copy(scores_hbm.at[pl.ds(ro, 8), pl.ds(co, TCOLS)], ts)
        pltpu.sync_copy(gumbel_hbm.at[pl.ds(ro, 8), pl.ds(co, TCOLS)], tg)
        process_slab(ts, tg, wid, co, TCOLS // L, masked=True)

    pltpu.sync_copy(acc_m, outm_hbm.at[wid])
    pltpu.sync_copy(acc_i, outi_hbm.at[wid])


def kernel(scores, gumbel):
    outm, outi = _sc_argmax(scores, gumbel)
    m = outm.transpose(1, 0, 2).reshape(R, NW * L)
    mi = outi.transpose(1, 0, 2).reshape(R, NW * L)
    gmax = jnp.max(m, axis=1, keepdims=True)
    gidx = jnp.min(jnp.where(m == gmax, mi, BIG_I32), axis=1)
    return gidx[:, None].astype(jnp.int32)
